# baseline (speedup 1.0000x reference)
import jax
import jax.numpy as jnp
from jax import lax
from jax.experimental import pallas as pl
from jax.experimental.pallas import tpu as pltpu
from jax.experimental.pallas import tpu_sc as plsc

N = 10000
E = 320000
D = 128

NC = 2
NS = 16
NW = NC * NS

CHUNK = 64
NCHUNKS = E // CHUNK
BASE_T = NCHUNKS // NW
REM = NCHUNKS % NW
N_PAD = 10112
ROWS_PER_TILE = N_PAD // NS


E_PER_W = E // NW


def _deg_body(e_hbm, deg_out, hist, idxr, sem_i):
    c = lax.axis_index("c")
    s = lax.axis_index("s")
    wid = c * NS + s
    ones16 = jnp.ones((16,), jnp.float32)

    base = pl.multiple_of(wid * E_PER_W, 8)
    icp = pltpu.async_copy(e_hbm.at[pl.ds(base, E_PER_W)], idxr, sem_i)

    def zero_body(i, _):
        hist[pl.ds(i * 16, 16)] = jnp.zeros((16,), jnp.float32)
        return _

    lax.fori_loop(0, N_PAD // 16, zero_body, None)
    icp.wait()

    def sc_body(k, _):
        iv = idxr[pl.ds(k * 16, 16)]
        plsc.addupdate_scatter(hist, [iv], ones16)
        return _

    lax.fori_loop(0, E_PER_W // 16, sc_body, None)
    pltpu.sync_copy(hist, deg_out.at[wid])


def _deg_sc(edge_index):
    mesh = plsc.VectorSubcoreMesh(core_axis_name="c", subcore_axis_name="s")
    return pl.kernel(
        _deg_body,
        out_type=jax.ShapeDtypeStruct((NW, N_PAD), jnp.float32),
        mesh=mesh,
        compiler_params=pltpu.CompilerParams(needs_layout_passes=False),
        scratch_types=[
            pltpu.VMEM((N_PAD,), jnp.float32),
            pltpu.VMEM((E_PER_W,), jnp.int32),
            pltpu.SemaphoreType.DMA,
        ],
    )(edge_index)


NBUF = 5
NIB = 6
GLEAD = 4
ILEAD = 5


def _scatter_body(y_hbm, e_hbm, zeros_hbm, out_hbm,
                  acc, sd, rows, sem_i, sem_g, sem_z):
    c = lax.axis_index("c")
    s = lax.axis_index("s")
    wid = c * NS + s
    t_w = BASE_T + jnp.where(wid < REM, 1, 0)
    rbase = pl.multiple_of(s * ROWS_PER_TILE, 8)

    zcp = pltpu.async_copy(zeros_hbm.at[pl.ds(rbase, ROWS_PER_TILE), :],
                           acc.at[pl.ds(rbase, ROWS_PER_TILE), :], sem_z)

    def start_idx(g):
        p = g % NIB
        base = (wid + g * NW) * CHUNK
        pltpu.async_copy(e_hbm.at[0, pl.ds(base, CHUNK)], sd.at[2 * p],
                         sem_i.at[2 * p])
        pltpu.async_copy(e_hbm.at[1, pl.ds(base, CHUNK)], sd.at[2 * p + 1],
                         sem_i.at[2 * p + 1])

    def wait_idx(g):
        p = g % NIB
        base = (wid + g * NW) * CHUNK
        pltpu.make_async_copy(e_hbm.at[0, pl.ds(base, CHUNK)], sd.at[2 * p],
                              sem_i.at[2 * p]).wait()
        pltpu.make_async_copy(e_hbm.at[1, pl.ds(base, CHUNK)],
                              sd.at[2 * p + 1], sem_i.at[2 * p + 1]).wait()

    def start_gather(g):
        pltpu.async_copy(y_hbm.at[sd.at[2 * (g % NIB)]], rows.at[g % NBUF],
                         sem_g.at[g % NBUF])

    def wait_gather(g):
        pltpu.make_async_copy(y_hbm.at[sd.at[2 * (g % NIB)]],
                              rows.at[g % NBUF], sem_g.at[g % NBUF]).wait()

    for k in range(ILEAD):
        start_idx(k)
    for k in range(GLEAD):
        wait_idx(k)
        start_gather(k)
    zcp.wait()
    plsc.subcore_barrier()

    def chunk_body(g, _):
        @pl.when(g + ILEAD < t_w)
        def _issue_idx():
            start_idx(g + ILEAD)

        @pl.when(g + GLEAD < t_w)
        def _issue_gather():
            wait_idx(g + GLEAD)
            start_gather(g + GLEAD)

        wait_gather(g)
        pltpu.sync_copy(rows.at[g % NBUF], acc.at[sd.at[2 * (g % NIB) + 1]],
                        add=True)
        return _

    lax.fori_loop(0, t_w, chunk_body, None)
    plsc.subcore_barrier()
    pltpu.sync_copy(acc.at[pl.ds(rbase, ROWS_PER_TILE), :],
                    out_hbm.at[c, pl.ds(rbase, ROWS_PER_TILE), :])


def _scatter_sc(y, edge_index, zeros_nd):
    mesh = plsc.VectorSubcoreMesh(core_axis_name="c", subcore_axis_name="s")
    return pl.kernel(
        _scatter_body,
        out_type=jax.ShapeDtypeStruct((NC, N_PAD, D), jnp.float32),
        mesh=mesh,
        compiler_params=pltpu.CompilerParams(needs_layout_passes=False),
        scratch_types=[
            pltpu.VMEM_SHARED((N_PAD, D), jnp.float32),
            pltpu.VMEM((2 * NIB, CHUNK), jnp.int32),
            pltpu.VMEM((NBUF, CHUNK, D), jnp.float32),
            pltpu.SemaphoreType.DMA((2 * NIB,)),
            pltpu.SemaphoreType.DMA((NBUF,)),
            pltpu.SemaphoreType.DMA,
        ],
    )(y, edge_index, zeros_nd)


def _tcmm_body(x_ref, w1_ref, u_ref):
    u = jnp.dot(x_ref[...], w1_ref[...],
                preferred_element_type=jnp.float32)
    u_ref[...] = jnp.concatenate(
        [u, jnp.zeros((N_PAD - N, D), jnp.float32)], axis=0)


def _tcmm(x, w1):
    return pl.pallas_call(
        _tcmm_body,
        out_shape=jax.ShapeDtypeStruct((N_PAD, D), jnp.float32),
    )(x, w1)


def _tc1_body(parts_ref, u_ref, disb_ref, y1_ref):
    deg = jnp.sum(parts_ref[...], axis=0) + 1.0
    dis = lax.rsqrt(deg)
    disb = jnp.broadcast_to(dis[:, None], (N_PAD, D))
    disb_ref[...] = disb
    y1_ref[...] = disb * u_ref[...]


def _tc1(deg_parts, u):
    return pl.pallas_call(
        _tc1_body,
        out_shape=(
            jax.ShapeDtypeStruct((N_PAD, D), jnp.float32),
            jax.ShapeDtypeStruct((N_PAD, D), jnp.float32),
        ),
    )(deg_parts, u)


def _bn_elu(g, gamma, beta, mask):
    gm = jnp.where(mask, g, 0.0)
    mean = jnp.sum(gm, axis=0) / N
    diff = g - mean
    var = jnp.sum(jnp.where(mask, diff * diff, 0.0), axis=0) / N
    xn = diff * lax.rsqrt(var + 1e-5)
    h = gamma * xn + beta
    return jnp.where(h > 0, h, jnp.exp(jnp.minimum(h, 0.0)) - 1.0)


def _tc2_body(sp_ref, y1_ref, disb_ref, w2_ref, b1_ref, g1_ref, be1_ref,
              y2_ref):
    mask = lax.broadcasted_iota(jnp.int32, (N_PAD, D), 0) < N
    ssum = sp_ref[0] + sp_ref[1] + y1_ref[...]
    g = disb_ref[...] * ssum + b1_ref[...]
    h1 = jnp.where(mask, _bn_elu(g, g1_ref[...], be1_ref[...], mask), 0.0)
    u2 = jnp.dot(h1, w2_ref[...], preferred_element_type=jnp.float32)
    y2_ref[...] = disb_ref[...] * u2


def _tc2(s1_parts, y1, disb, w2, b1r, g1r, be1r):
    return pl.pallas_call(
        _tc2_body,
        out_shape=jax.ShapeDtypeStruct((N_PAD, D), jnp.float32),
    )(s1_parts, y1, disb, w2, b1r, g1r, be1r)


def _tc3_body(sp_ref, y2_ref, disb_ref, b2_ref, g2_ref, be2_ref, out_ref):
    mask = lax.broadcasted_iota(jnp.int32, (N_PAD, D), 0) < N
    ssum = sp_ref[0] + sp_ref[1] + y2_ref[...]
    g = disb_ref[...] * ssum + b2_ref[...]
    h2 = _bn_elu(g, g2_ref[...], be2_ref[...], mask)
    out_ref[...] = h2[:N]


def _tc3(s2_parts, y2, disb, b2r, g2r, be2r):
    return pl.pallas_call(
        _tc3_body,
        out_shape=jax.ShapeDtypeStruct((N, D), jnp.float32),
    )(s2_parts, y2, disb, b2r, g2r, be2r)


@jax.jit
def kernel(x, edge_index, W1, b1, gamma1, beta1, W2, b2, gamma2, beta2):
    zeros_nd = jnp.zeros((N_PAD, D), jnp.float32)
    b1r = b1[None, :]
    g1r = gamma1[None, :]
    be1r = beta1[None, :]
    b2r = b2[None, :]
    g2r = gamma2[None, :]
    be2r = beta2[None, :]

    deg_parts = _deg_sc(edge_index[1])
    u1 = _tcmm(x, W1)
    disb, y1 = _tc1(deg_parts, u1)
    s1_parts = _scatter_sc(y1, edge_index, zeros_nd)
    y2 = _tc2(s1_parts, y1, disb, W2, b1r, g1r, be1r)
    s2_parts = _scatter_sc(y2, edge_index, zeros_nd)
    return _tc3(s2_parts, y2, disb, b2r, g2r, be2r)


# speedup vs baseline: 1.0418x; 1.0418x over previous
import jax
import jax.numpy as jnp
from jax import lax
from jax.experimental import pallas as pl
from jax.experimental.pallas import tpu as pltpu
from jax.experimental.pallas import tpu_sc as plsc

N = 10000
E = 320000
D = 128

NC = 2
NS = 16
NW = NC * NS

CHUNK = 64
NCHUNKS = E // CHUNK
BASE_T = NCHUNKS // NW
REM = NCHUNKS % NW
N_PAD = 10112
ROWS_PER_TILE = N_PAD // NS


E_PER_W = E // NW


def _deg_body(e_hbm, deg_out, hist, idxr, sem_i):
    c = lax.axis_index("c")
    s = lax.axis_index("s")
    wid = c * NS + s
    ones16 = jnp.ones((16,), jnp.float32)

    base = pl.multiple_of(E + wid * E_PER_W, 8)
    icp = pltpu.async_copy(e_hbm.at[pl.ds(base, E_PER_W)], idxr, sem_i)

    def zero_body(i, _):
        hist[pl.ds(i * 16, 16)] = jnp.zeros((16,), jnp.float32)
        return _

    lax.fori_loop(0, N_PAD // 16, zero_body, None)
    icp.wait()

    def sc_body(k, _):
        iv = idxr[pl.ds(k * 16, 16)]
        plsc.addupdate_scatter(hist, [iv], ones16)
        return _

    lax.fori_loop(0, E_PER_W // 16, sc_body, None)
    pltpu.sync_copy(hist, deg_out.at[wid])


def _deg_sc(edge_index):
    mesh = plsc.VectorSubcoreMesh(core_axis_name="c", subcore_axis_name="s")
    return pl.kernel(
        _deg_body,
        out_type=jax.ShapeDtypeStruct((NW, N_PAD), jnp.float32),
        mesh=mesh,
        compiler_params=pltpu.CompilerParams(needs_layout_passes=False),
        scratch_types=[
            pltpu.VMEM((N_PAD,), jnp.float32),
            pltpu.VMEM((E_PER_W,), jnp.int32),
            pltpu.SemaphoreType.DMA,
        ],
    )(edge_index)


NBUF = 5
NIB = 6
GLEAD = 4
ILEAD = 5


def _scatter_body(y_hbm, e_hbm, zeros_hbm, out_hbm,
                  acc, sd, rows, sem_i, sem_g, sem_z):
    c = lax.axis_index("c")
    s = lax.axis_index("s")
    wid = c * NS + s
    t_w = BASE_T + jnp.where(wid < REM, 1, 0)
    rbase = pl.multiple_of(s * ROWS_PER_TILE, 8)

    zcp = pltpu.async_copy(zeros_hbm.at[pl.ds(rbase, ROWS_PER_TILE), :],
                           acc.at[pl.ds(rbase, ROWS_PER_TILE), :], sem_z)

    def start_idx(g):
        p = g % NIB
        base = (wid + g * NW) * CHUNK
        pltpu.async_copy(e_hbm.at[0, pl.ds(base, CHUNK)], sd.at[2 * p],
                         sem_i.at[2 * p])
        pltpu.async_copy(e_hbm.at[1, pl.ds(base, CHUNK)], sd.at[2 * p + 1],
                         sem_i.at[2 * p + 1])

    def wait_idx(g):
        p = g % NIB
        base = (wid + g * NW) * CHUNK
        pltpu.make_async_copy(e_hbm.at[0, pl.ds(base, CHUNK)], sd.at[2 * p],
                              sem_i.at[2 * p]).wait()
        pltpu.make_async_copy(e_hbm.at[1, pl.ds(base, CHUNK)],
                              sd.at[2 * p + 1], sem_i.at[2 * p + 1]).wait()

    def start_gather(g):
        pltpu.async_copy(y_hbm.at[sd.at[2 * (g % NIB)]], rows.at[g % NBUF],
                         sem_g.at[g % NBUF])

    def wait_gather(g):
        pltpu.make_async_copy(y_hbm.at[sd.at[2 * (g % NIB)]],
                              rows.at[g % NBUF], sem_g.at[g % NBUF]).wait()

    for k in range(ILEAD):
        start_idx(k)
    for k in range(GLEAD):
        wait_idx(k)
        start_gather(k)
    zcp.wait()
    plsc.subcore_barrier()

    def chunk_body(g, _):
        @pl.when(g + ILEAD < t_w)
        def _issue_idx():
            start_idx(g + ILEAD)

        @pl.when(g + GLEAD < t_w)
        def _issue_gather():
            wait_idx(g + GLEAD)
            start_gather(g + GLEAD)

        wait_gather(g)
        pltpu.sync_copy(rows.at[g % NBUF], acc.at[sd.at[2 * (g % NIB) + 1]],
                        add=True)
        return _

    lax.fori_loop(0, t_w, chunk_body, None)
    plsc.subcore_barrier()
    pltpu.sync_copy(acc.at[pl.ds(rbase, ROWS_PER_TILE), :],
                    out_hbm.at[c, pl.ds(rbase, ROWS_PER_TILE), :])


def _scatter_sc(y, edge_index, zeros_nd):
    mesh = plsc.VectorSubcoreMesh(core_axis_name="c", subcore_axis_name="s")
    return pl.kernel(
        _scatter_body,
        out_type=jax.ShapeDtypeStruct((NC, N_PAD, D), jnp.float32),
        mesh=mesh,
        compiler_params=pltpu.CompilerParams(needs_layout_passes=False),
        scratch_types=[
            pltpu.VMEM_SHARED((N_PAD, D), jnp.float32),
            pltpu.VMEM((2 * NIB, CHUNK), jnp.int32),
            pltpu.VMEM((NBUF, CHUNK, D), jnp.float32),
            pltpu.SemaphoreType.DMA((2 * NIB,)),
            pltpu.SemaphoreType.DMA((NBUF,)),
            pltpu.SemaphoreType.DMA,
        ],
    )(y, edge_index, zeros_nd)


def _tcmm_body(x_ref, w1_ref, u_ref):
    u = jnp.dot(x_ref[...], w1_ref[...],
                preferred_element_type=jnp.float32)
    u_ref[...] = jnp.concatenate(
        [u, jnp.zeros((N_PAD - N, D), jnp.float32)], axis=0)


def _tcmm(x, w1):
    return pl.pallas_call(
        _tcmm_body,
        out_shape=jax.ShapeDtypeStruct((N_PAD, D), jnp.float32),
    )(x, w1)


def _tc1_body(parts_ref, u_ref, disb_ref, y1_ref):
    deg = jnp.sum(parts_ref[...], axis=0) + 1.0
    dis = lax.rsqrt(deg)
    disb = jnp.broadcast_to(dis[:, None], (N_PAD, D))
    disb_ref[...] = disb
    y1_ref[...] = disb * u_ref[...]


def _tc1(deg_parts, u):
    return pl.pallas_call(
        _tc1_body,
        out_shape=(
            jax.ShapeDtypeStruct((N_PAD, D), jnp.float32),
            jax.ShapeDtypeStruct((N_PAD, D), jnp.float32),
        ),
    )(deg_parts, u)


def _bn_elu(g, gamma, beta, mask):
    gm = jnp.where(mask, g, 0.0)
    mean = jnp.sum(gm, axis=0) / N
    diff = g - mean
    var = jnp.sum(jnp.where(mask, diff * diff, 0.0), axis=0) / N
    xn = diff * lax.rsqrt(var + 1e-5)
    h = gamma * xn + beta
    return jnp.where(h > 0, h, jnp.exp(jnp.minimum(h, 0.0)) - 1.0)


def _tc2_body(sp_ref, y1_ref, disb_ref, w2_ref, b1_ref, g1_ref, be1_ref,
              y2_ref):
    mask = lax.broadcasted_iota(jnp.int32, (N_PAD, D), 0) < N
    ssum = sp_ref[0] + sp_ref[1] + y1_ref[...]
    g = disb_ref[...] * ssum + b1_ref[...]
    h1 = jnp.where(mask, _bn_elu(g, g1_ref[...], be1_ref[...], mask), 0.0)
    u2 = jnp.dot(h1, w2_ref[...], preferred_element_type=jnp.float32)
    y2_ref[...] = disb_ref[...] * u2


def _tc2(s1_parts, y1, disb, w2, b1r, g1r, be1r):
    return pl.pallas_call(
        _tc2_body,
        out_shape=jax.ShapeDtypeStruct((N_PAD, D), jnp.float32),
    )(s1_parts, y1, disb, w2, b1r, g1r, be1r)


def _tc3_body(sp_ref, y2_ref, disb_ref, b2_ref, g2_ref, be2_ref, out_ref):
    mask = lax.broadcasted_iota(jnp.int32, (N_PAD, D), 0) < N
    ssum = sp_ref[0] + sp_ref[1] + y2_ref[...]
    g = disb_ref[...] * ssum + b2_ref[...]
    h2 = _bn_elu(g, g2_ref[...], be2_ref[...], mask)
    out_ref[...] = h2[:N]


def _tc3(s2_parts, y2, disb, b2r, g2r, be2r):
    return pl.pallas_call(
        _tc3_body,
        out_shape=jax.ShapeDtypeStruct((N, D), jnp.float32),
    )(s2_parts, y2, disb, b2r, g2r, be2r)


@jax.jit
def kernel(x, edge_index, W1, b1, gamma1, beta1, W2, b2, gamma2, beta2):
    zeros_nd = jnp.zeros((N_PAD, D), jnp.float32)
    b1r = b1[None, :]
    g1r = gamma1[None, :]
    be1r = beta1[None, :]
    b2r = b2[None, :]
    g2r = gamma2[None, :]
    be2r = beta2[None, :]

    deg_parts = _deg_sc(edge_index.reshape(2 * E))
    u1 = _tcmm(x, W1)
    disb, y1 = _tc1(deg_parts, u1)
    s1_parts = _scatter_sc(y1, edge_index, zeros_nd)
    y2 = _tc2(s1_parts, y1, disb, W2, b1r, g1r, be1r)
    s2_parts = _scatter_sc(y2, edge_index, zeros_nd)
    return _tc3(s2_parts, y2, disb, b2r, g2r, be2r)
